# baseline (speedup 1.0000x reference)
import functools

import jax
import jax.numpy as jnp
from jax import lax
from jax.experimental import pallas as pl
from jax.experimental.pallas import tpu as pltpu
from jax.experimental.pallas import tpu_sc as plsc

N = 10000
NP = 10240
E = 320000
D = 128
H = 128
C = 64
BLK = 2048
NC = 2
NS = 16
NW = NC * NS
EPW = E // NW
K = 48
NG = K // 16
NCHUNK = EPW // K + 1
NDUP = NCHUNK * K - EPW
LAST_OFF = EPW - K
RPT = NP // NS
ZR = 16
LANES = 16

_GATHER_DNUMS = lax.GatherDimensionNumbers(
    offset_dims=(), collapsed_slice_dims=(0,), start_index_map=(0,))


def _lane_gather(v, idx):
    return lax.gather(v, idx[:, None], _GATHER_DNUMS, (1,),
                      mode=lax.GatherScatterMode.PROMISE_IN_BOUNDS)


def _splat(v, i):
    return _lane_gather(v, jnp.full((LANES,), i, dtype=jnp.int32))


def _sc_layer_body(h_hbm, hnb_hbm, inv_hbm, src_hbm, dst_hbm,
                   num_hbm, den_hbm,
                   acc_sh, sidx0, sidx1, didx0, didx1, dsc0, dsc1,
                   hs0, hs1, hd0, hd1,
                   invl, denl, si0, si1, sg0, sg1, ss0, ss1):
    cid = lax.axis_index("c")
    sid = lax.axis_index("s")
    wid = sid * NC + cid
    ebase = wid * EPW
    tbase = pl.multiple_of(sid * RPT, 8)

    sidx = (sidx0, sidx1)
    didx = (didx0, didx1)
    dsc = (dsc0, dsc1)
    hs = (hs0, hs1)
    hd = (hd0, hd1)
    si = (si0, si1)
    sg = (sg0, sg1)
    ss = (ss0, ss1)

    pltpu.sync_copy(inv_hbm, invl)
    zv = jnp.zeros((LANES,), jnp.float32)

    def zhd(i, carry):
        for j in range(D // LANES):
            hd0[i, pl.ds(j * LANES, LANES)] = zv
        return carry

    lax.fori_loop(0, K, zhd, 0)

    def zden(i, carry):
        denl[pl.ds(i * LANES, LANES)] = zv
        return carry

    lax.fori_loop(0, NP // LANES, zden, 0)

    def zacc(i, carry):
        pltpu.sync_copy(
            hd0, acc_sh.at[pl.ds(pl.multiple_of(tbase + i * K, 8), K)])
        return carry

    lax.fori_loop(0, RPT // K, zacc, 0)
    pltpu.sync_copy(hd0.at[pl.ds(0, RPT - (RPT // K) * K)],
                    acc_sh.at[pl.ds(pl.multiple_of(
                        tbase + (RPT // K) * K, 8), RPT - (RPT // K) * K)])
    plsc.subcore_barrier()

    lane_iota = lax.iota(jnp.int32, LANES)
    bfly = [jnp.bitwise_xor(lane_iota, m) for m in (8, 4, 2, 1)]
    mask0 = lane_iota == 0

    def off_of(c):
        return pl.multiple_of(ebase + jnp.minimum(c * K, LAST_OFF), 8)

    def issue_idx(c, s):
        o = off_of(c)
        pltpu.async_copy(src_hbm.at[pl.ds(o, K)], sidx[s], si[s])
        pltpu.async_copy(dst_hbm.at[pl.ds(o, K)], didx[s], si[s])

    def wait_idx(c, s):
        o = off_of(c)
        pltpu.make_async_copy(src_hbm.at[pl.ds(o, K)], sidx[s], si[s]).wait()
        pltpu.make_async_copy(dst_hbm.at[pl.ds(o, K)], didx[s], si[s]).wait()

    def issue_gather(s):
        pltpu.async_copy(h_hbm.at[sidx[s]], hs[s], sg[s])
        pltpu.async_copy(hnb_hbm.at[didx[s]], hd[s], sg[s])

    def wait_gather(s):
        pltpu.make_async_copy(h_hbm.at[sidx[s]], hs[s], sg[s]).wait()
        pltpu.make_async_copy(hnb_hbm.at[didx[s]], hd[s], sg[s]).wait()

    def issue_scatter(s):
        for g in range(NG):
            dsc[s][pl.ds(g * LANES, LANES)] = didx[s][pl.ds(g * LANES, LANES)]
        pltpu.async_copy(hs[s], acc_sh.at[dsc[s]], ss[s], add=True)

    def wait_scatter(s):
        pltpu.make_async_copy(hs[s], acc_sh.at[dsc[s]], ss[s]).wait()

    def edge_group(s, g, zf):
        gb = g * LANES
        sv = sidx[s][pl.ds(gb, LANES)]
        dv = didx[s][pl.ds(gb, LANES)]
        inv16 = plsc.load_gather(invl, [sv])
        for kk in range(LANES):
            k = gb + kk
            a = [hs[s][k, pl.ds(j * LANES, LANES)] for j in range(D // LANES)]
            p = [aj * hd[s][k, pl.ds(j * LANES, LANES)]
                 for j, aj in enumerate(a)]
            while len(p) > 1:
                p = [x + y for x, y in zip(p[::2], p[1::2])]
            acc = p[0]
            for bidx in bfly:
                acc = acc + _lane_gather(acc, bidx)
            ex = jnp.exp(acc * _splat(inv16, kk)) * zf
            for j, aj in enumerate(a):
                hs[s][k, pl.ds(j * LANES, LANES)] = ex * aj
            plsc.addupdate_scatter(denl, [_splat(dv, kk)], ex, mask=mask0)

    def compute_chunk(s, ndup_groups=0):
        if ndup_groups:
            for g in range(NG):
                edge_group(s, g, 0.0 if g < ndup_groups else 1.0)
        else:
            def group(g, gcarry):
                edge_group(s, g, 1.0)
                return gcarry
            lax.fori_loop(0, NG, group, 0)

    def chunk_body(c, s, first=False, tail=False):
        os = 1 - s
        if not first:
            wait_scatter(os)
        if not tail:
            wait_idx(c + 1, os)
            issue_gather(os)
        wait_gather(s)
        compute_chunk(s, NDUP // LANES if tail else 0)
        issue_scatter(s)
        if not tail and not (isinstance(c, int) and c + 2 > NCHUNK - 1):
            issue_idx(c + 2, s)

    issue_idx(0, 0)
    issue_idx(1, 1)
    wait_idx(0, 0)
    issue_gather(0)

    chunk_body(0, 0, first=True)

    def pair(p, carry):
        chunk_body(2 * p + 1, 1)
        chunk_body(2 * p + 2, 0)
        return carry

    lax.fori_loop(0, (NCHUNK - 3) // 2, pair, 0)
    chunk_body(NCHUNK - 2, 1)
    chunk_body(NCHUNK - 1, 0, tail=True)
    wait_scatter(0)
    plsc.subcore_barrier()

    pltpu.sync_copy(acc_sh.at[pl.ds(tbase, RPT)],
                    num_hbm.at[cid, pl.ds(tbase, RPT)])
    pltpu.sync_copy(denl, den_hbm.at[cid, sid])


_sc_layer = pl.kernel(
    _sc_layer_body,
    out_type=[
        jax.ShapeDtypeStruct((NC, NP, D), jnp.float32),
        jax.ShapeDtypeStruct((NC, NS, NP), jnp.float32),
    ],
    mesh=plsc.VectorSubcoreMesh(core_axis_name="c", subcore_axis_name="s"),
    compiler_params=pltpu.CompilerParams(needs_layout_passes=False),
    scratch_types=[
        pltpu.VMEM_SHARED((NP, D), jnp.float32),
        pltpu.VMEM((K,), jnp.int32),
        pltpu.VMEM((K,), jnp.int32),
        pltpu.VMEM((K,), jnp.int32),
        pltpu.VMEM((K,), jnp.int32),
        pltpu.VMEM((K,), jnp.int32),
        pltpu.VMEM((K,), jnp.int32),
        pltpu.VMEM((K, D), jnp.float32),
        pltpu.VMEM((K, D), jnp.float32),
        pltpu.VMEM((K, D), jnp.float32),
        pltpu.VMEM((K, D), jnp.float32),
        pltpu.VMEM((NP,), jnp.float32),
        pltpu.VMEM((NP,), jnp.float32),
        pltpu.SemaphoreType.DMA,
        pltpu.SemaphoreType.DMA,
        pltpu.SemaphoreType.DMA,
        pltpu.SemaphoreType.DMA,
        pltpu.SemaphoreType.DMA,
        pltpu.SemaphoreType.DMA,
    ],
)


def _proj_body(x_ref, w_ref, b_ref, beta_ref, h_ref, hnb_ref, inv_ref):
    h = jnp.maximum(
        jnp.dot(x_ref[...], w_ref[...], preferred_element_type=jnp.float32)
        + b_ref[...],
        0.0,
    )
    inv = 1.0 / (jnp.sqrt(jnp.sum(h * h, axis=1, keepdims=True)) + 1e-12)
    h_ref[...] = h
    hnb_ref[...] = h * (inv * beta_ref[0])
    inv_ref[...] = inv


def _proj(x, W1, b1, beta):
    return pl.pallas_call(
        _proj_body,
        grid=(NP // BLK,),
        in_specs=[
            pl.BlockSpec((BLK, D), lambda i: (i, 0)),
            pl.BlockSpec((D, H), lambda i: (0, 0)),
            pl.BlockSpec((1, H), lambda i: (0, 0)),
            pl.BlockSpec(memory_space=pltpu.SMEM),
        ],
        out_specs=[
            pl.BlockSpec((BLK, H), lambda i: (i, 0)),
            pl.BlockSpec((BLK, H), lambda i: (i, 0)),
            pl.BlockSpec((BLK, 1), lambda i: (i, 0)),
        ],
        out_shape=[
            jax.ShapeDtypeStruct((NP, H), jnp.float32),
            jax.ShapeDtypeStruct((NP, H), jnp.float32),
            jax.ShapeDtypeStruct((NP, 1), jnp.float32),
        ],
    )(x, W1, b1.reshape(1, H), beta)


def _agg(p_ref, d_ref):
    acc = p_ref[0] + p_ref[1]
    den = jnp.sum(d_ref[...], axis=(0, 1))[:, None]
    return acc / (den + 1e-12)


def _combine_body(p_ref, d_ref, beta_ref, h_ref, hnb_ref, inv_ref):
    out = _agg(p_ref, d_ref)
    inv = 1.0 / (jnp.sqrt(jnp.sum(out * out, axis=1, keepdims=True)) + 1e-12)
    h_ref[...] = out
    hnb_ref[...] = out * (inv * beta_ref[0])
    inv_ref[...] = inv


def _combine(num, den, beta):
    return pl.pallas_call(
        _combine_body,
        grid=(NP // BLK,),
        in_specs=[
            pl.BlockSpec((NC, BLK, D), lambda i: (0, i, 0)),
            pl.BlockSpec((NC, NS, BLK), lambda i: (0, 0, i)),
            pl.BlockSpec(memory_space=pltpu.SMEM),
        ],
        out_specs=[
            pl.BlockSpec((BLK, H), lambda i: (i, 0)),
            pl.BlockSpec((BLK, H), lambda i: (i, 0)),
            pl.BlockSpec((BLK, 1), lambda i: (i, 0)),
        ],
        out_shape=[
            jax.ShapeDtypeStruct((NP, H), jnp.float32),
            jax.ShapeDtypeStruct((NP, H), jnp.float32),
            jax.ShapeDtypeStruct((NP, 1), jnp.float32),
        ],
    )(num, den, beta)


def _cls_body(p_ref, d_ref, w_ref, b_ref, o_ref):
    out = _agg(p_ref, d_ref)
    o_ref[...] = (
        jnp.dot(out, w_ref[...], preferred_element_type=jnp.float32)
        + b_ref[...]
    )


def _cls(num, den, W2, b2):
    return pl.pallas_call(
        _cls_body,
        grid=(NP // BLK,),
        in_specs=[
            pl.BlockSpec((NC, BLK, D), lambda i: (0, i, 0)),
            pl.BlockSpec((NC, NS, BLK), lambda i: (0, 0, i)),
            pl.BlockSpec((H, C), lambda i: (0, 0)),
            pl.BlockSpec((1, C), lambda i: (0, 0)),
        ],
        out_specs=pl.BlockSpec((BLK, C), lambda i: (i, 0)),
        out_shape=jax.ShapeDtypeStruct((NP, C), jnp.float32),
    )(num, den, W2, b2.reshape(1, C))


def kernel(features, edge_index, W1, b1, betas, W2, b2):
    src = edge_index[0]
    dst = edge_index[1]
    beta0 = betas[0].reshape(1)
    beta1 = betas[1].reshape(1)
    h, hnb, inv = _proj(features, W1, b1, beta0)
    num, den = _sc_layer(h, hnb, inv.reshape(NP), src, dst)
    h, hnb, inv = _combine(num, den, beta1)
    num, den = _sc_layer(h, hnb, inv.reshape(NP), src, dst)
    return _cls(num, den, W2, b2)[:N]


# speedup vs baseline: 1.2326x; 1.2326x over previous
import functools

import jax
import jax.numpy as jnp
from jax import lax
from jax.experimental import pallas as pl
from jax.experimental.pallas import tpu as pltpu
from jax.experimental.pallas import tpu_sc as plsc

N = 10000
NP = 10240
E = 320000
D = 128
H = 128
C = 64
BLK = 2048
NC = 2
NS = 16
NW = NC * NS
EPW = E // NW
K = 32
NG = K // 16
NCHUNK = EPW // K + 1
NDUP = NCHUNK * K - EPW
LAST_OFF = EPW - K
RPT = NP // NS
ZR = 16
LANES = 16

_GATHER_DNUMS = lax.GatherDimensionNumbers(
    offset_dims=(), collapsed_slice_dims=(0,), start_index_map=(0,))


def _lane_gather(v, idx):
    return lax.gather(v, idx[:, None], _GATHER_DNUMS, (1,),
                      mode=lax.GatherScatterMode.PROMISE_IN_BOUNDS)


def _splat(v, i):
    return _lane_gather(v, jnp.full((LANES,), i, dtype=jnp.int32))


def _sc_layer_body(h_hbm, hnb_hbm, inv_hbm, src_hbm, dst_hbm,
                   num_hbm, den_hbm,
                   acc_sh, sidx0, sidx1, didx0, didx1, dsc0, dsc1,
                   hs0, hs1, hd0, hd1,
                   invl, denl, dotb, si0, si1, sg0, sg1, ss0, ss1):
    cid = lax.axis_index("c")
    sid = lax.axis_index("s")
    wid = sid * NC + cid
    ebase = wid * EPW
    tbase = pl.multiple_of(sid * RPT, 8)

    sidx = (sidx0, sidx1)
    didx = (didx0, didx1)
    dsc = (dsc0, dsc1)
    hs = (hs0, hs1)
    hd = (hd0, hd1)
    si = (si0, si1)
    sg = (sg0, sg1)
    ss = (ss0, ss1)

    pltpu.sync_copy(inv_hbm, invl)
    zv = jnp.zeros((LANES,), jnp.float32)

    def zhd(i, carry):
        for j in range(D // LANES):
            hd0[i, pl.ds(j * LANES, LANES)] = zv
        return carry

    lax.fori_loop(0, K, zhd, 0)

    def zden(i, carry):
        denl[pl.ds(i * LANES, LANES)] = zv
        return carry

    lax.fori_loop(0, NP // LANES, zden, 0)

    def zacc(i, carry):
        pltpu.sync_copy(
            hd0, acc_sh.at[pl.ds(pl.multiple_of(tbase + i * K, 8), K)])
        return carry

    lax.fori_loop(0, RPT // K, zacc, 0)
    if RPT % K:
        pltpu.sync_copy(hd0.at[pl.ds(0, RPT % K)],
                        acc_sh.at[pl.ds(pl.multiple_of(
                            tbase + (RPT // K) * K, 8), RPT % K)])
    plsc.subcore_barrier()

    lane_iota = lax.iota(jnp.int32, LANES)
    mask0 = lane_iota == 0

    def off_of(c):
        return pl.multiple_of(ebase + jnp.minimum(c * K, LAST_OFF), 8)

    def issue_idx(c, s):
        o = off_of(c)
        pltpu.async_copy(src_hbm.at[pl.ds(o, K)], sidx[s], si[s])
        pltpu.async_copy(dst_hbm.at[pl.ds(o, K)], didx[s], si[s])

    def wait_idx(c, s):
        o = off_of(c)
        pltpu.make_async_copy(src_hbm.at[pl.ds(o, K)], sidx[s], si[s]).wait()
        pltpu.make_async_copy(dst_hbm.at[pl.ds(o, K)], didx[s], si[s]).wait()

    def issue_gather(s):
        pltpu.async_copy(h_hbm.at[sidx[s]], hs[s], sg[s])
        pltpu.async_copy(hnb_hbm.at[didx[s]], hd[s], sg[s])

    def wait_gather(s):
        pltpu.make_async_copy(h_hbm.at[sidx[s]], hs[s], sg[s]).wait()
        pltpu.make_async_copy(hnb_hbm.at[didx[s]], hd[s], sg[s]).wait()

    def issue_scatter(s):
        for g in range(NG):
            dsc[s][pl.ds(g * LANES, LANES)] = didx[s][pl.ds(g * LANES, LANES)]
        pltpu.async_copy(hs[s], acc_sh.at[dsc[s]], ss[s], add=True)

    def wait_scatter(s):
        pltpu.make_async_copy(hs[s], acc_sh.at[dsc[s]], ss[s]).wait()

    def edge_group(s, g, zf):
        gb = g * LANES
        sv = sidx[s][pl.ds(gb, LANES)]
        dv = didx[s][pl.ds(gb, LANES)]
        inv16 = plsc.load_gather(invl, [sv])
        for kk in range(LANES):
            k = gb + kk
            acc = hs[s][k, pl.ds(0, LANES)] * hd[s][k, pl.ds(0, LANES)]
            for j in range(1, D // LANES):
                acc = acc + (hs[s][k, pl.ds(j * LANES, LANES)]
                             * hd[s][k, pl.ds(j * LANES, LANES)])
            dotb[kk, pl.ds(0, LANES)] = acc
        tot = plsc.load_gather(
            dotb, [lane_iota, jnp.full((LANES,), 0, jnp.int32)])
        for j in range(1, LANES):
            tot = tot + plsc.load_gather(
                dotb, [lane_iota, jnp.full((LANES,), j, jnp.int32)])
        ex16 = jnp.exp(tot * inv16) * zf
        for kk in range(LANES):
            k = gb + kk
            ex = _splat(ex16, kk)
            for j in range(D // LANES):
                hs[s][k, pl.ds(j * LANES, LANES)] = (
                    ex * hs[s][k, pl.ds(j * LANES, LANES)])
            plsc.addupdate_scatter(denl, [_splat(dv, kk)], ex, mask=mask0)

    def compute_chunk(s, ndup_groups=0):
        if ndup_groups:
            for g in range(NG):
                edge_group(s, g, 0.0 if g < ndup_groups else 1.0)
        else:
            def group(g, gcarry):
                edge_group(s, g, 1.0)
                return gcarry
            lax.fori_loop(0, NG, group, 0)

    def chunk_body(c, s, first=False, tail=False):
        os = 1 - s
        if not first:
            wait_scatter(os)
        if not tail:
            wait_idx(c + 1, os)
            issue_gather(os)
        wait_gather(s)
        compute_chunk(s, NDUP // LANES if tail else 0)
        issue_scatter(s)
        if not tail and not (isinstance(c, int) and c + 2 > NCHUNK - 1):
            issue_idx(c + 2, s)

    issue_idx(0, 0)
    issue_idx(1, 1)
    wait_idx(0, 0)
    issue_gather(0)

    chunk_body(0, 0, first=True)

    def pair(p, carry):
        chunk_body(2 * p + 1, 1)
        chunk_body(2 * p + 2, 0)
        return carry

    lax.fori_loop(0, (NCHUNK - 3) // 2, pair, 0)
    chunk_body(NCHUNK - 2, 1)
    chunk_body(NCHUNK - 1, 0, tail=True)
    wait_scatter(0)
    plsc.subcore_barrier()

    pltpu.sync_copy(acc_sh.at[pl.ds(tbase, RPT)],
                    num_hbm.at[cid, pl.ds(tbase, RPT)])
    pltpu.sync_copy(denl, den_hbm.at[cid, sid])


_sc_layer = pl.kernel(
    _sc_layer_body,
    out_type=[
        jax.ShapeDtypeStruct((NC, NP, D), jnp.float32),
        jax.ShapeDtypeStruct((NC, NS, NP), jnp.float32),
    ],
    mesh=plsc.VectorSubcoreMesh(core_axis_name="c", subcore_axis_name="s"),
    compiler_params=pltpu.CompilerParams(needs_layout_passes=False),
    scratch_types=[
        pltpu.VMEM_SHARED((NP, D), jnp.float32),
        pltpu.VMEM((K,), jnp.int32),
        pltpu.VMEM((K,), jnp.int32),
        pltpu.VMEM((K,), jnp.int32),
        pltpu.VMEM((K,), jnp.int32),
        pltpu.VMEM((K,), jnp.int32),
        pltpu.VMEM((K,), jnp.int32),
        pltpu.VMEM((K, D), jnp.float32),
        pltpu.VMEM((K, D), jnp.float32),
        pltpu.VMEM((K, D), jnp.float32),
        pltpu.VMEM((K, D), jnp.float32),
        pltpu.VMEM((NP,), jnp.float32),
        pltpu.VMEM((NP,), jnp.float32),
        pltpu.VMEM((LANES, LANES), jnp.float32),
        pltpu.SemaphoreType.DMA,
        pltpu.SemaphoreType.DMA,
        pltpu.SemaphoreType.DMA,
        pltpu.SemaphoreType.DMA,
        pltpu.SemaphoreType.DMA,
        pltpu.SemaphoreType.DMA,
    ],
)


def _proj_body(x_ref, w_ref, b_ref, beta_ref, h_ref, hnb_ref, inv_ref):
    h = jnp.maximum(
        jnp.dot(x_ref[...], w_ref[...], preferred_element_type=jnp.float32)
        + b_ref[...],
        0.0,
    )
    inv = 1.0 / (jnp.sqrt(jnp.sum(h * h, axis=1, keepdims=True)) + 1e-12)
    h_ref[...] = h
    hnb_ref[...] = h * (inv * beta_ref[0])
    inv_ref[...] = inv


def _proj(x, W1, b1, beta):
    return pl.pallas_call(
        _proj_body,
        grid=(NP // BLK,),
        in_specs=[
            pl.BlockSpec((BLK, D), lambda i: (i, 0)),
            pl.BlockSpec((D, H), lambda i: (0, 0)),
            pl.BlockSpec((1, H), lambda i: (0, 0)),
            pl.BlockSpec(memory_space=pltpu.SMEM),
        ],
        out_specs=[
            pl.BlockSpec((BLK, H), lambda i: (i, 0)),
            pl.BlockSpec((BLK, H), lambda i: (i, 0)),
            pl.BlockSpec((BLK, 1), lambda i: (i, 0)),
        ],
        out_shape=[
            jax.ShapeDtypeStruct((NP, H), jnp.float32),
            jax.ShapeDtypeStruct((NP, H), jnp.float32),
            jax.ShapeDtypeStruct((NP, 1), jnp.float32),
        ],
    )(x, W1, b1.reshape(1, H), beta)


def _agg(p_ref, d_ref):
    acc = p_ref[0] + p_ref[1]
    den = jnp.sum(d_ref[...], axis=(0, 1))[:, None]
    return acc / (den + 1e-12)


def _combine_body(p_ref, d_ref, beta_ref, h_ref, hnb_ref, inv_ref):
    out = _agg(p_ref, d_ref)
    inv = 1.0 / (jnp.sqrt(jnp.sum(out * out, axis=1, keepdims=True)) + 1e-12)
    h_ref[...] = out
    hnb_ref[...] = out * (inv * beta_ref[0])
    inv_ref[...] = inv


def _combine(num, den, beta):
    return pl.pallas_call(
        _combine_body,
        grid=(NP // BLK,),
        in_specs=[
            pl.BlockSpec((NC, BLK, D), lambda i: (0, i, 0)),
            pl.BlockSpec((NC, NS, BLK), lambda i: (0, 0, i)),
            pl.BlockSpec(memory_space=pltpu.SMEM),
        ],
        out_specs=[
            pl.BlockSpec((BLK, H), lambda i: (i, 0)),
            pl.BlockSpec((BLK, H), lambda i: (i, 0)),
            pl.BlockSpec((BLK, 1), lambda i: (i, 0)),
        ],
        out_shape=[
            jax.ShapeDtypeStruct((NP, H), jnp.float32),
            jax.ShapeDtypeStruct((NP, H), jnp.float32),
            jax.ShapeDtypeStruct((NP, 1), jnp.float32),
        ],
    )(num, den, beta)


def _cls_body(p_ref, d_ref, w_ref, b_ref, o_ref):
    out = _agg(p_ref, d_ref)
    o_ref[...] = (
        jnp.dot(out, w_ref[...], preferred_element_type=jnp.float32)
        + b_ref[...]
    )


def _cls(num, den, W2, b2):
    return pl.pallas_call(
        _cls_body,
        grid=(NP // BLK,),
        in_specs=[
            pl.BlockSpec((NC, BLK, D), lambda i: (0, i, 0)),
            pl.BlockSpec((NC, NS, BLK), lambda i: (0, 0, i)),
            pl.BlockSpec((H, C), lambda i: (0, 0)),
            pl.BlockSpec((1, C), lambda i: (0, 0)),
        ],
        out_specs=pl.BlockSpec((BLK, C), lambda i: (i, 0)),
        out_shape=jax.ShapeDtypeStruct((NP, C), jnp.float32),
    )(num, den, W2, b2.reshape(1, C))


def kernel(features, edge_index, W1, b1, betas, W2, b2):
    src = edge_index[0]
    dst = edge_index[1]
    beta0 = betas[0].reshape(1)
    beta1 = betas[1].reshape(1)
    h, hnb, inv = _proj(features, W1, b1, beta0)
    num, den = _sc_layer(h, hnb, inv.reshape(NP), src, dst)
    h, hnb, inv = _combine(num, den, beta1)
    num, den = _sc_layer(h, hnb, inv.reshape(NP), src, dst)
    return _cls(num, den, W2, b2)[:N]


# speedup vs baseline: 1.3440x; 1.0904x over previous
import functools

import jax
import jax.numpy as jnp
from jax import lax
from jax.experimental import pallas as pl
from jax.experimental.pallas import tpu as pltpu
from jax.experimental.pallas import tpu_sc as plsc

N = 10000
NP = 10240
E = 320000
D = 128
H = 128
C = 64
BLK = 2048
NC = 2
NS = 16
NW = NC * NS
EPW = E // NW
K = 32
NG = K // 16
NCHUNK = EPW // K + 1
NDUP = NCHUNK * K - EPW
LAST_OFF = EPW - K
RPT = NP // NS
ZR = 16
LANES = 16

_GATHER_DNUMS = lax.GatherDimensionNumbers(
    offset_dims=(), collapsed_slice_dims=(0,), start_index_map=(0,))


def _lane_gather(v, idx):
    return lax.gather(v, idx[:, None], _GATHER_DNUMS, (1,),
                      mode=lax.GatherScatterMode.PROMISE_IN_BOUNDS)


def _splat(v, i):
    return _lane_gather(v, jnp.full((LANES,), i, dtype=jnp.int32))


def _sc_layer_body(h_hbm, hnb_hbm, inv_hbm, src_hbm, dst_hbm,
                   num_hbm, den_hbm,
                   acc_sh, sidx0, sidx1, didx0, didx1, dsc0, dsc1,
                   hs0, hs1, hd0, hd1,
                   invl, denl, dotb, si0, si1, sg0, sg1, ss0, ss1):
    cid = lax.axis_index("c")
    sid = lax.axis_index("s")
    wid = sid * NC + cid
    ebase = wid * EPW
    tbase = pl.multiple_of(sid * RPT, 8)

    sidx = (sidx0, sidx1)
    didx = (didx0, didx1)
    dsc = (dsc0, dsc1)
    hs = (hs0, hs1)
    hd = (hd0, hd1)
    si = (si0, si1)
    sg = (sg0, sg1)
    ss = (ss0, ss1)

    pltpu.sync_copy(inv_hbm, invl)
    zv = jnp.zeros((LANES,), jnp.float32)

    def zhd(i, carry):
        for j in range(D // LANES):
            hd0[i, pl.ds(j * LANES, LANES)] = zv
        return carry

    lax.fori_loop(0, K, zhd, 0)

    def zden(i, carry):
        denl[pl.ds(i * LANES, LANES)] = zv
        return carry

    lax.fori_loop(0, NP // LANES, zden, 0)

    def zacc(i, carry):
        pltpu.sync_copy(
            hd0, acc_sh.at[pl.ds(pl.multiple_of(tbase + i * K, 8), K)])
        return carry

    lax.fori_loop(0, RPT // K, zacc, 0)
    if RPT % K:
        pltpu.sync_copy(hd0.at[pl.ds(0, RPT % K)],
                        acc_sh.at[pl.ds(pl.multiple_of(
                            tbase + (RPT // K) * K, 8), RPT % K)])
    plsc.subcore_barrier()

    lane_iota = lax.iota(jnp.int32, LANES)
    mask0 = lane_iota == 0

    def off_of(c):
        return pl.multiple_of(ebase + jnp.minimum(c * K, LAST_OFF), 8)

    def issue_idx(c, s):
        o = off_of(c)
        pltpu.async_copy(src_hbm.at[pl.ds(o, K)], sidx[s], si[s])
        pltpu.async_copy(dst_hbm.at[pl.ds(o, K)], didx[s], si[s])

    def wait_idx(c, s):
        o = off_of(c)
        pltpu.make_async_copy(src_hbm.at[pl.ds(o, K)], sidx[s], si[s]).wait()
        pltpu.make_async_copy(dst_hbm.at[pl.ds(o, K)], didx[s], si[s]).wait()

    def issue_gather(s):
        pltpu.async_copy(h_hbm.at[sidx[s]], hs[s], sg[s])
        pltpu.async_copy(hnb_hbm.at[didx[s]], hd[s], sg[s])

    def wait_gather(s):
        pltpu.make_async_copy(h_hbm.at[sidx[s]], hs[s], sg[s]).wait()
        pltpu.make_async_copy(hnb_hbm.at[didx[s]], hd[s], sg[s]).wait()

    def issue_scatter(s):
        for g in range(NG):
            dsc[s][pl.ds(g * LANES, LANES)] = didx[s][pl.ds(g * LANES, LANES)]
        pltpu.async_copy(hs[s], acc_sh.at[dsc[s]], ss[s], add=True)

    def wait_scatter(s):
        pltpu.make_async_copy(hs[s], acc_sh.at[dsc[s]], ss[s]).wait()

    def edge_group(s, g, zf):
        gb = g * LANES
        sv = sidx[s][pl.ds(gb, LANES)]
        dv = didx[s][pl.ds(gb, LANES)]
        inv16 = plsc.load_gather(invl, [sv])
        for kk in range(LANES):
            k = gb + kk
            acc = hs[s][k, pl.ds(0, LANES)] * hd[s][k, pl.ds(0, LANES)]
            for j in range(1, D // LANES):
                acc = acc + (hs[s][k, pl.ds(j * LANES, LANES)]
                             * hd[s][k, pl.ds(j * LANES, LANES)])
            dotb[kk, pl.ds(0, LANES)] = acc
        tot = plsc.load_gather(
            dotb, [lane_iota, jnp.full((LANES,), 0, jnp.int32)])
        for j in range(1, LANES):
            tot = tot + plsc.load_gather(
                dotb, [lane_iota, jnp.full((LANES,), j, jnp.int32)])
        ex16 = jnp.exp(tot * inv16) * zf
        plsc.addupdate_scatter(denl, [dv], ex16)
        for kk in range(LANES):
            k = gb + kk
            ex = _splat(ex16, kk)
            for j in range(D // LANES):
                hs[s][k, pl.ds(j * LANES, LANES)] = (
                    ex * hs[s][k, pl.ds(j * LANES, LANES)])

    def compute_chunk(s, ndup_groups=0):
        if ndup_groups:
            for g in range(NG):
                edge_group(s, g, 0.0 if g < ndup_groups else 1.0)
        else:
            def group(g, gcarry):
                edge_group(s, g, 1.0)
                return gcarry
            lax.fori_loop(0, NG, group, 0)

    def chunk_body(c, s, first=False, tail=False):
        os = 1 - s
        if not first:
            wait_scatter(os)
        if not tail:
            wait_idx(c + 1, os)
            issue_gather(os)
        wait_gather(s)
        compute_chunk(s, NDUP // LANES if tail else 0)
        issue_scatter(s)
        if not tail and not (isinstance(c, int) and c + 2 > NCHUNK - 1):
            issue_idx(c + 2, s)

    issue_idx(0, 0)
    issue_idx(1, 1)
    wait_idx(0, 0)
    issue_gather(0)

    chunk_body(0, 0, first=True)

    def pair(p, carry):
        chunk_body(2 * p + 1, 1)
        chunk_body(2 * p + 2, 0)
        return carry

    lax.fori_loop(0, (NCHUNK - 3) // 2, pair, 0)
    chunk_body(NCHUNK - 2, 1)
    chunk_body(NCHUNK - 1, 0, tail=True)
    wait_scatter(0)
    plsc.subcore_barrier()

    pltpu.sync_copy(acc_sh.at[pl.ds(tbase, RPT)],
                    num_hbm.at[cid, pl.ds(tbase, RPT)])
    pltpu.sync_copy(denl, den_hbm.at[cid, sid])


_sc_layer = pl.kernel(
    _sc_layer_body,
    out_type=[
        jax.ShapeDtypeStruct((NC, NP, D), jnp.float32),
        jax.ShapeDtypeStruct((NC, NS, NP), jnp.float32),
    ],
    mesh=plsc.VectorSubcoreMesh(core_axis_name="c", subcore_axis_name="s"),
    compiler_params=pltpu.CompilerParams(needs_layout_passes=False),
    scratch_types=[
        pltpu.VMEM_SHARED((NP, D), jnp.float32),
        pltpu.VMEM((K,), jnp.int32),
        pltpu.VMEM((K,), jnp.int32),
        pltpu.VMEM((K,), jnp.int32),
        pltpu.VMEM((K,), jnp.int32),
        pltpu.VMEM((K,), jnp.int32),
        pltpu.VMEM((K,), jnp.int32),
        pltpu.VMEM((K, D), jnp.float32),
        pltpu.VMEM((K, D), jnp.float32),
        pltpu.VMEM((K, D), jnp.float32),
        pltpu.VMEM((K, D), jnp.float32),
        pltpu.VMEM((NP,), jnp.float32),
        pltpu.VMEM((NP,), jnp.float32),
        pltpu.VMEM((LANES, LANES), jnp.float32),
        pltpu.SemaphoreType.DMA,
        pltpu.SemaphoreType.DMA,
        pltpu.SemaphoreType.DMA,
        pltpu.SemaphoreType.DMA,
        pltpu.SemaphoreType.DMA,
        pltpu.SemaphoreType.DMA,
    ],
)


def _proj_body(x_ref, w_ref, b_ref, beta_ref, h_ref, hnb_ref, inv_ref):
    h = jnp.maximum(
        jnp.dot(x_ref[...], w_ref[...], preferred_element_type=jnp.float32)
        + b_ref[...],
        0.0,
    )
    inv = 1.0 / (jnp.sqrt(jnp.sum(h * h, axis=1, keepdims=True)) + 1e-12)
    h_ref[...] = h
    hnb_ref[...] = h * (inv * beta_ref[0])
    inv_ref[...] = inv


def _proj(x, W1, b1, beta):
    return pl.pallas_call(
        _proj_body,
        grid=(NP // BLK,),
        in_specs=[
            pl.BlockSpec((BLK, D), lambda i: (i, 0)),
            pl.BlockSpec((D, H), lambda i: (0, 0)),
            pl.BlockSpec((1, H), lambda i: (0, 0)),
            pl.BlockSpec(memory_space=pltpu.SMEM),
        ],
        out_specs=[
            pl.BlockSpec((BLK, H), lambda i: (i, 0)),
            pl.BlockSpec((BLK, H), lambda i: (i, 0)),
            pl.BlockSpec((BLK, 1), lambda i: (i, 0)),
        ],
        out_shape=[
            jax.ShapeDtypeStruct((NP, H), jnp.float32),
            jax.ShapeDtypeStruct((NP, H), jnp.float32),
            jax.ShapeDtypeStruct((NP, 1), jnp.float32),
        ],
    )(x, W1, b1.reshape(1, H), beta)


def _agg(p_ref, d_ref):
    acc = p_ref[0] + p_ref[1]
    den = jnp.sum(d_ref[...], axis=(0, 1))[:, None]
    return acc / (den + 1e-12)


def _combine_body(p_ref, d_ref, beta_ref, h_ref, hnb_ref, inv_ref):
    out = _agg(p_ref, d_ref)
    inv = 1.0 / (jnp.sqrt(jnp.sum(out * out, axis=1, keepdims=True)) + 1e-12)
    h_ref[...] = out
    hnb_ref[...] = out * (inv * beta_ref[0])
    inv_ref[...] = inv


def _combine(num, den, beta):
    return pl.pallas_call(
        _combine_body,
        grid=(NP // BLK,),
        in_specs=[
            pl.BlockSpec((NC, BLK, D), lambda i: (0, i, 0)),
            pl.BlockSpec((NC, NS, BLK), lambda i: (0, 0, i)),
            pl.BlockSpec(memory_space=pltpu.SMEM),
        ],
        out_specs=[
            pl.BlockSpec((BLK, H), lambda i: (i, 0)),
            pl.BlockSpec((BLK, H), lambda i: (i, 0)),
            pl.BlockSpec((BLK, 1), lambda i: (i, 0)),
        ],
        out_shape=[
            jax.ShapeDtypeStruct((NP, H), jnp.float32),
            jax.ShapeDtypeStruct((NP, H), jnp.float32),
            jax.ShapeDtypeStruct((NP, 1), jnp.float32),
        ],
    )(num, den, beta)


def _cls_body(p_ref, d_ref, w_ref, b_ref, o_ref):
    out = _agg(p_ref, d_ref)
    o_ref[...] = (
        jnp.dot(out, w_ref[...], preferred_element_type=jnp.float32)
        + b_ref[...]
    )


def _cls(num, den, W2, b2):
    return pl.pallas_call(
        _cls_body,
        grid=(NP // BLK,),
        in_specs=[
            pl.BlockSpec((NC, BLK, D), lambda i: (0, i, 0)),
            pl.BlockSpec((NC, NS, BLK), lambda i: (0, 0, i)),
            pl.BlockSpec((H, C), lambda i: (0, 0)),
            pl.BlockSpec((1, C), lambda i: (0, 0)),
        ],
        out_specs=pl.BlockSpec((BLK, C), lambda i: (i, 0)),
        out_shape=jax.ShapeDtypeStruct((NP, C), jnp.float32),
    )(num, den, W2, b2.reshape(1, C))


def kernel(features, edge_index, W1, b1, betas, W2, b2):
    src = edge_index[0]
    dst = edge_index[1]
    beta0 = betas[0].reshape(1)
    beta1 = betas[1].reshape(1)
    h, hnb, inv = _proj(features, W1, b1, beta0)
    num, den = _sc_layer(h, hnb, inv.reshape(NP), src, dst)
    h, hnb, inv = _combine(num, den, beta1)
    num, den = _sc_layer(h, hnb, inv.reshape(NP), src, dst)
    return _cls(num, den, W2, b2)[:N]
